# Initial kernel scaffold; baseline (speedup 1.0000x reference)
import jax
import jax.numpy as jnp
from jax.experimental import pallas as pl


def kernel(x, edge_index1, edge_index2, edge_weight1, edge_weight2, W1, W2):
    raise NotImplementedError("write your pallas kernel here")


# speedup vs baseline: 4.2294x; 4.2294x over previous
import functools

import jax
import jax.numpy as jnp
from jax import lax
from jax.experimental import pallas as pl
from jax.experimental.pallas import tpu as pltpu
from jax.experimental.pallas import tpu_sc as plsc

N_NODES = 10000
NFEAT = 128
N_CORES = 2
N_SUBCORES = 16
N_WORKERS = N_CORES * N_SUBCORES
CHUNK = 80
LANES = 16
F_CHUNKS = NFEAT // LANES
ROWS_PER_TILE = 624
TAIL_ROW0 = N_SUBCORES * ROWS_PER_TILE
TAIL_ROWS = N_NODES - TAIL_ROW0


def _sc_aggregate(h, src, dst, w):
    n_edges = src.shape[0]
    e_per_worker = n_edges // N_WORKERS
    n_chunks = e_per_worker // CHUNK
    assert e_per_worker % CHUNK == 0

    mesh = plsc.VectorSubcoreMesh(core_axis_name="c", subcore_axis_name="s")

    @functools.partial(
        pl.kernel,
        mesh=mesh,
        out_type=jax.ShapeDtypeStruct((N_CORES, N_NODES, NFEAT), jnp.float32),
        scratch_types=[
            pltpu.VMEM_SHARED((N_NODES, NFEAT), jnp.float32),
            pltpu.VMEM((CHUNK,), jnp.int32),
            pltpu.VMEM((CHUNK,), jnp.int32),
            pltpu.VMEM((CHUNK,), jnp.float32),
            pltpu.VMEM((CHUNK, NFEAT), jnp.float32),
            pltpu.SemaphoreType.DMA,
        ],
        compiler_params=pltpu.CompilerParams(use_tc_tiling_on_sc=False),
    )
    def agg(h_hbm, src_hbm, dst_hbm, w_hbm, out_hbm,
            acc, src_v, dst_v, w_v, msgs, sem):
        c = lax.axis_index("c")
        s = lax.axis_index("s")
        zero16 = jnp.zeros((LANES,), jnp.float32)

        def zero_row(r, carry):
            for f in range(F_CHUNKS):
                msgs[r, pl.ds(f * LANES, LANES)] = zero16
            return carry

        lax.fori_loop(0, CHUNK, zero_row, 0)
        row0 = s * ROWS_PER_TILE
        for j in range(ROWS_PER_TILE // CHUNK):
            pltpu.sync_copy(msgs, acc.at[pl.ds(row0 + j * CHUNK, CHUNK)])
        rem0 = (ROWS_PER_TILE // CHUNK) * CHUNK
        rem = ROWS_PER_TILE - rem0
        if rem:
            pltpu.sync_copy(msgs.at[pl.ds(0, rem)],
                            acc.at[pl.ds(row0 + rem0, rem)])

        @pl.when(s == N_SUBCORES - 1)
        def _zero_tail():
            pltpu.sync_copy(msgs.at[pl.ds(0, TAIL_ROWS)],
                            acc.at[pl.ds(TAIL_ROW0, TAIL_ROWS)])

        plsc.subcore_barrier()

        base = (c * N_SUBCORES + s) * e_per_worker

        def chunk_body(k, carry):
            off = pl.multiple_of(base + k * CHUNK, 8)
            pltpu.sync_copy(src_hbm.at[pl.ds(off, CHUNK)], src_v)
            pltpu.sync_copy(dst_hbm.at[pl.ds(off, CHUNK)], dst_v)
            pltpu.sync_copy(w_hbm.at[pl.ds(off, CHUNK)], w_v)
            pltpu.async_copy(h_hbm.at[src_v], msgs, sem).wait()

            def scale(g, cc):
                wv16 = w_v[pl.ds(g * LANES, LANES)]
                for r in range(LANES):
                    wr = wv16[r]
                    e = g * LANES + r
                    for f in range(F_CHUNKS):
                        sl = pl.ds(f * LANES, LANES)
                        msgs[e, sl] = msgs[e, sl] * wr
                return cc

            lax.fori_loop(0, CHUNK // LANES, scale, 0)
            pltpu.sync_copy(msgs, acc.at[dst_v], add=True)
            return carry

        lax.fori_loop(0, n_chunks, chunk_body, 0)

        plsc.subcore_barrier()
        pltpu.sync_copy(acc.at[pl.ds(row0, ROWS_PER_TILE)],
                        out_hbm.at[c, pl.ds(row0, ROWS_PER_TILE)])

        @pl.when(s == N_SUBCORES - 1)
        def _copy_tail():
            pltpu.sync_copy(acc.at[pl.ds(TAIL_ROW0, TAIL_ROWS)],
                            out_hbm.at[c, pl.ds(TAIL_ROW0, TAIL_ROWS)])

    return agg(h, src, dst, w)


def _combine_matmul(parts, W, relu):
    rows_blk = 1000

    def mm(p_ref, w_ref, o_ref):
        a = p_ref[0] + p_ref[1]
        y = jnp.dot(a, w_ref[...], preferred_element_type=jnp.float32)
        if relu:
            y = jnp.maximum(y, 0.0)
        o_ref[...] = y

    return pl.pallas_call(
        mm,
        grid=(N_NODES // rows_blk,),
        in_specs=[
            pl.BlockSpec((N_CORES, rows_blk, NFEAT), lambda i: (0, i, 0)),
            pl.BlockSpec((NFEAT, NFEAT), lambda i: (0, 0)),
        ],
        out_specs=pl.BlockSpec((rows_blk, NFEAT), lambda i: (i, 0)),
        out_shape=jax.ShapeDtypeStruct((N_NODES, NFEAT), jnp.float32),
    )(parts, W)


def kernel(x, edge_index1, edge_index2, edge_weight1, edge_weight2, W1, W2):
    src1 = edge_index1[0].astype(jnp.int32)
    dst1 = edge_index1[1].astype(jnp.int32)
    src2 = edge_index2[0].astype(jnp.int32)
    dst2 = edge_index2[1].astype(jnp.int32)

    p1 = _sc_aggregate(x, src1, dst1, edge_weight1)
    h = _combine_matmul(p1, W1, relu=True)
    p2 = _sc_aggregate(h, src2, dst2, edge_weight2)
    return _combine_matmul(p2, W2, relu=False)


# speedup vs baseline: 7.1783x; 1.6972x over previous
import functools

import jax
import jax.numpy as jnp
from jax import lax
from jax.experimental import pallas as pl
from jax.experimental.pallas import tpu as pltpu
from jax.experimental.pallas import tpu_sc as plsc

N_NODES = 10000
NFEAT = 128
N_CORES = 2
N_SUBCORES = 16
N_WORKERS = N_CORES * N_SUBCORES
LANES = 16
F_CHUNKS = NFEAT // LANES
ROWS_PER_TILE = 624
TAIL_ROW0 = N_SUBCORES * ROWS_PER_TILE
TAIL_ROWS = N_NODES - TAIL_ROW0

CHUNK = 96
G_CHUNKS = CHUNK // LANES
N_CHUNKS = 105
S_PHASE = 35
N_PHASES = N_CHUNKS // S_PHASE
E_PER_WORKER = N_CHUNKS * CHUNK


def _pack_edges(src, dst, w):
    n_real = src.shape[0] // N_WORKERS

    def shape(a):
        a = a.reshape(N_WORKERS, n_real)
        a = jnp.pad(a, ((0, 0), (0, E_PER_WORKER - n_real)))
        return a.reshape(N_WORKERS, N_CHUNKS, CHUNK)

    return jnp.stack([shape(src), shape(dst)], axis=2), shape(w)


def _sc_aggregate(h, packed, packed_w):
    mesh = plsc.VectorSubcoreMesh(core_axis_name="c", subcore_axis_name="s")

    @functools.partial(
        pl.kernel,
        mesh=mesh,
        out_type=jax.ShapeDtypeStruct((N_CORES, N_NODES, NFEAT), jnp.float32),
        scratch_types=[
            pltpu.VMEM_SHARED((N_NODES, NFEAT), jnp.float32),
            pltpu.VMEM((S_PHASE, 2, CHUNK), jnp.int32),
            pltpu.VMEM((S_PHASE, CHUNK), jnp.float32),
            pltpu.VMEM((CHUNK, NFEAT), jnp.float32),
            pltpu.VMEM((CHUNK, NFEAT), jnp.float32),
            pltpu.VMEM((CHUNK, NFEAT), jnp.float32),
            pltpu.SemaphoreType.DMA,
            pltpu.SemaphoreType.DMA,
            pltpu.SemaphoreType.DMA,
            pltpu.SemaphoreType.DMA,
            pltpu.SemaphoreType.DMA,
            pltpu.SemaphoreType.DMA,
        ],
        compiler_params=pltpu.CompilerParams(use_tc_tiling_on_sc=False),
    )
    def agg(h_hbm, packed_hbm, packedw_hbm, out_hbm,
            acc, stage, stage_w, m0, m1, m2, g0, g1, g2, s0, s1, s2):
        c = lax.axis_index("c")
        s = lax.axis_index("s")
        wid = c * N_SUBCORES + s
        msgs = (m0, m1, m2)
        gsem = (g0, g1, g2)
        ssem = (s0, s1, s2)
        zero16 = jnp.zeros((LANES,), jnp.float32)

        def zero_row(r, carry):
            for f in range(F_CHUNKS):
                m0[r, pl.ds(f * LANES, LANES)] = zero16
            return carry

        lax.fori_loop(0, CHUNK, zero_row, 0)
        row0 = s * ROWS_PER_TILE
        for jz in range(ROWS_PER_TILE // CHUNK):
            pltpu.sync_copy(m0, acc.at[pl.ds(row0 + jz * CHUNK, CHUNK)])
        rem0 = (ROWS_PER_TILE // CHUNK) * CHUNK
        rem = ROWS_PER_TILE - rem0
        if rem:
            pltpu.sync_copy(m0.at[pl.ds(0, rem)],
                            acc.at[pl.ds(row0 + rem0, rem)])

        @pl.when(s == N_SUBCORES - 1)
        def _zero_tail():
            pltpu.sync_copy(m0.at[pl.ds(0, TAIL_ROWS)],
                            acc.at[pl.ds(TAIL_ROW0, TAIL_ROWS)])

        plsc.subcore_barrier()

        def issue_gather(j, b):
            pltpu.async_copy(h_hbm.at[stage.at[j, 0]], msgs[b], gsem[b])

        def wait_gather(j, b):
            pltpu.make_async_copy(h_hbm.at[stage.at[j, 0]],
                                  msgs[b], gsem[b]).wait()

        def issue_scatter(j, b):
            pltpu.async_copy(msgs[b], acc.at[stage.at[j, 1]], ssem[b],
                             add=True)

        def wait_scatter(j, b):
            pltpu.make_async_copy(msgs[b], acc.at[stage.at[j, 1]],
                                  ssem[b]).wait()

        def scale(j, b):
            mb = msgs[b]

            def scale_g(g, carry):
                wv = stage_w[j, pl.ds(g * LANES, LANES)]
                for r2 in range(LANES):
                    e = g * LANES + r2
                    wr = wv[r2]
                    for f in range(F_CHUNKS):
                        sl = pl.ds(f * LANES, LANES)
                        mb[e, sl] = mb[e, sl] * wr
                return carry

            lax.fori_loop(0, G_CHUNKS, scale_g, 0)

        def process(j, b):
            wait_gather(j, b)
            scale(j, b)
            issue_scatter(j, b)

        def phase_body(ph, carry):
            pltpu.sync_copy(packed_hbm.at[wid, pl.ds(ph * S_PHASE, S_PHASE)],
                            stage)
            pltpu.sync_copy(packedw_hbm.at[wid, pl.ds(ph * S_PHASE, S_PHASE)],
                            stage_w)
            issue_gather(0, 0)
            issue_gather(1, 1)
            process(0, 0)
            issue_gather(2, 2)
            process(1, 1)
            wait_scatter(0, 0)
            issue_gather(3, 0)
            process(2, 2)
            wait_scatter(1, 1)
            issue_gather(4, 1)

            def triple(t, cc):
                for r in range(3):
                    j = 3 + 3 * t + r
                    b = r
                    process(j, b)
                    nb = (r + 2) % 3
                    wait_scatter(j - 1, nb)
                    issue_gather(j + 2, nb)
                return cc

            lax.fori_loop(0, (S_PHASE - 5) // 3, triple, 0)

            process(S_PHASE - 2, 0)
            process(S_PHASE - 1, 1)
            wait_scatter(S_PHASE - 3, 2)
            wait_scatter(S_PHASE - 2, 0)
            wait_scatter(S_PHASE - 1, 1)
            return carry

        lax.fori_loop(0, N_PHASES, phase_body, 0)

        plsc.subcore_barrier()
        pltpu.sync_copy(acc.at[pl.ds(row0, ROWS_PER_TILE)],
                        out_hbm.at[c, pl.ds(row0, ROWS_PER_TILE)])

        @pl.when(s == N_SUBCORES - 1)
        def _copy_tail():
            pltpu.sync_copy(acc.at[pl.ds(TAIL_ROW0, TAIL_ROWS)],
                            out_hbm.at[c, pl.ds(TAIL_ROW0, TAIL_ROWS)])

    return agg(h, packed, packed_w)


def _combine_matmul(parts, W, relu):
    rows_blk = 1000

    def mm(p_ref, w_ref, o_ref):
        a = p_ref[0] + p_ref[1]
        y = jnp.dot(a, w_ref[...], preferred_element_type=jnp.float32)
        if relu:
            y = jnp.maximum(y, 0.0)
        o_ref[...] = y

    return pl.pallas_call(
        mm,
        grid=(N_NODES // rows_blk,),
        in_specs=[
            pl.BlockSpec((N_CORES, rows_blk, NFEAT), lambda i: (0, i, 0)),
            pl.BlockSpec((NFEAT, NFEAT), lambda i: (0, 0)),
        ],
        out_specs=pl.BlockSpec((rows_blk, NFEAT), lambda i: (i, 0)),
        out_shape=jax.ShapeDtypeStruct((N_NODES, NFEAT), jnp.float32),
    )(parts, W)


def kernel(x, edge_index1, edge_index2, edge_weight1, edge_weight2, W1, W2):
    packed1, pw1 = _pack_edges(edge_index1[0].astype(jnp.int32),
                               edge_index1[1].astype(jnp.int32), edge_weight1)
    packed2, pw2 = _pack_edges(edge_index2[0].astype(jnp.int32),
                               edge_index2[1].astype(jnp.int32), edge_weight2)

    p1 = _sc_aggregate(x, packed1, pw1)
    h = _combine_matmul(p1, W1, relu=True)
    p2 = _sc_aggregate(h, packed2, pw2)
    return _combine_matmul(p2, W2, relu=False)
